# baseline (speedup 1.0000x reference)
import jax
import jax.numpy as jnp
from jax.experimental import pallas as pl

_N0, _N1, _N2 = 10000, 30000, 20000
_D = 128
_SLOPE = 0.2


def _mm_body(x_ref, w_ref, o_ref):
    o_ref[...] = jnp.dot(x_ref[...], w_ref[...],
                         preferred_element_type=jnp.float32)


def _mm(x, w, bn=400):
    n, d = x.shape
    c = w.shape[1]
    return pl.pallas_call(
        _mm_body,
        grid=(n // bn,),
        in_specs=[pl.BlockSpec((bn, d), lambda i: (i, 0)),
                  pl.BlockSpec((d, c), lambda i: (0, 0))],
        out_specs=pl.BlockSpec((bn, c), lambda i: (i, 0)),
        out_shape=jax.ShapeDtypeStruct((n, c), jnp.float32),
    )(x, w)


def _row_norm(vals, rows, n):
    s = jax.ops.segment_sum(vals, rows, num_segments=n)
    return vals / s[rows]


def _hbs(x, idx, W, a, n):
    msg = _mm(x, W)
    i, j = idx[0], idx[1]
    z = jnp.concatenate([msg[i], msg[j]], axis=1)
    e = jax.nn.leaky_relu(z @ a, _SLOPE)[:, 0]
    att = _row_norm(e, i, n)
    return jax.ops.segment_sum(att[:, None] * msg[j], i, num_segments=n)


def _hbns(x_s, x_t, idx, w_s, w_t, a, n_t, n_s):
    s_msg = _mm(x_s, w_s)
    t_msg = _mm(x_t, w_t)
    ti, sj = idx[0], idx[1]
    e = jax.nn.leaky_relu(jnp.concatenate([s_msg[sj], t_msg[ti]], axis=1) @ a, _SLOPE)[:, 0]
    f = jax.nn.leaky_relu(jnp.concatenate([t_msg[ti], s_msg[sj]], axis=1) @ a, _SLOPE)[:, 0]
    e = _row_norm(e, ti, n_t)
    f = _row_norm(f, sj, n_s)
    msg_on_target = jax.ops.segment_sum(e[:, None] * s_msg[sj], ti, num_segments=n_t)
    msg_on_source = jax.ops.segment_sum(f[:, None] * t_msg[ti], sj, num_segments=n_s)
    return msg_on_source, msg_on_target


def kernel(x_0, x_1, x_2, adjacency_0, adjacency_1, coadjacency_2,
           incidence_1, incidence_2, params):
    p = params
    x_0_to_0 = _hbs(x_0, adjacency_0, p["hbs_0_l1_w"], p["hbs_0_l1_a"], _N0)
    x_0_to_1, x_1_to_0 = _hbns(x_1, x_0, incidence_1, p["hbns_01_l1_ws"], p["hbns_01_l1_wt"], p["hbns_01_l1_a"], _N0, _N1)
    x_1_to_2, x_2_to_1 = _hbns(x_2, x_1, incidence_2, p["hbns_12_l1_ws"], p["hbns_12_l1_wt"], p["hbns_12_l1_a"], _N1, _N2)
    x_0_l1 = x_0_to_0 + x_1_to_0
    x_1_l1 = x_0_to_1 + x_2_to_1
    x_2_l1 = x_1_to_2
    x_0_to_0 = _hbs(x_0_l1, adjacency_0, p["hbs_0_l2_w"], p["hbs_0_l2_a"], _N0)
    x_0_to_1, x_1_to_0 = _hbns(x_1_l1, x_0_l1, incidence_1, p["hbns_01_l2_ws"], p["hbns_01_l2_wt"], p["hbns_01_l2_a"], _N0, _N1)
    x_1_to_1 = _hbs(x_1_l1, adjacency_1, p["hbs_1_l2_w"], p["hbs_1_l2_a"], _N1)
    x_1_to_2, x_2_to_1 = _hbns(x_2_l1, x_1_l1, incidence_2, p["hbns_12_l2_ws"], p["hbns_12_l2_wt"], p["hbns_12_l2_a"], _N1, _N2)
    x_2_to_2 = _hbs(x_2_l1, coadjacency_2, p["hbs_2_l2_w"], p["hbs_2_l2_a"], _N2)
    x_0_l2 = x_0_to_0 + x_1_to_0
    x_1_l2 = x_0_to_1 + x_1_to_1 + x_2_to_1
    x_2_l2 = x_1_to_2 + x_2_to_2
    return x_0_l2, x_1_l2, x_2_l2
